# baseline (speedup 1.0000x reference)
import jax
import jax.numpy as jnp
from jax import lax
from jax.experimental import pallas as pl
from jax.experimental.pallas import tpu as pltpu
from jax.experimental.pallas import tpu_sc as plsc

N = 10000
D = 128
E = 320000
PS = 10240
PR = PS // 128

NC = 2
NS = 16
NW = NC * NS
EPW = E // NW
CH = 80
NCHUNK = EPW // CH
EPP = NCHUNK * CH
NH = N // 2
NACC = NH + 8
GROW = NH
ZRH = 112
NZH = 4
ZSH = 304

_SENT = -3.0e38


def _score_body(x_ref, p_ref, u_ref, pn_ref):
    p = p_ref[...]
    pn_ref[...] = jnp.sqrt(
        jnp.sum(p * p, axis=(0, 1), keepdims=True)) + 1e-16
    u_ref[...] = jnp.dot(x_ref[...], p, preferred_element_type=jnp.float32)


def _k2a(X, p2):
    return pl.pallas_call(
        _score_body,
        out_shape=[
            jax.ShapeDtypeStruct((N, 1), jnp.float32),
            jax.ShapeDtypeStruct((1, 1), jnp.float32),
        ],
    )(X, p2)


def _remap(cidx_v, cadj_v, j, lo):
    for g in range(CH // 16):
        v = cidx_v[pl.ds(j * CH + g * 16, 16)] - lo
        ok = (v >= 0) & (v < NH)
        cadj_v[pl.ds(g * 16, 16)] = jnp.where(ok, v, GROW)


def _deg_body(cole_hbm, zeros_hbm, out_hbm, cidx_v, cadj_v, ones_v,
              zbuf_v, shared_deg):
    c = lax.axis_index("c")
    s = lax.axis_index("s")
    lo = c * NH
    base = s * ZSH
    one16 = jnp.ones((16,), jnp.float32)
    for i in range(CH):
        for t in range(D // 16):
            ones_v[i, pl.ds(t * 16, 16)] = one16
    pltpu.sync_copy(zeros_hbm, zbuf_v)
    for k in range(NZH):
        pltpu.sync_copy(zbuf_v, shared_deg.at[pl.ds(base + k * ZRH, ZRH)])
    plsc.subcore_barrier()

    def body(j, carry):
        _remap(cidx_v, cadj_v, j, lo)
        pltpu.sync_copy(ones_v, shared_deg.at[cadj_v], add=True)
        return carry

    for seg in (s, s + NS):
        pltpu.sync_copy(cole_hbm.at[seg], cidx_v)
        lax.fori_loop(0, NCHUNK, body, 0)
    plsc.subcore_barrier()
    for k in range(NZH):
        pltpu.sync_copy(shared_deg.at[pl.ds(base + k * ZRH, ZRH)],
                        out_hbm.at[c, pl.ds(base + k * ZRH, ZRH)])


def _k1(cole, zeros_deg):
    mesh = plsc.VectorSubcoreMesh(core_axis_name="c", subcore_axis_name="s")
    return pl.kernel(
        _deg_body,
        out_type=jax.ShapeDtypeStruct((NC, NACC, D), jnp.float32),
        mesh=mesh,
        scratch_types=[
            pltpu.VMEM((EPP,), jnp.int32),
            pltpu.VMEM((CH,), jnp.int32),
            pltpu.VMEM((CH, D), jnp.float32),
            pltpu.VMEM((ZRH, D), jnp.float32),
            pltpu.VMEM_SHARED((NACC, D), jnp.float32),
        ],
    )(cole, zeros_deg)


def _evolve_body(u2_ref, x_ref, pn_ref, h0_ref, wihT_ref, whhT_ref,
                 bih_ref, bhh_ref, xw_ref, xt_ref):
    pn = pn_ref[0, 0]
    fi = (lax.broadcasted_iota(jnp.int32, (PR, 128), 0) * 128 +
          lax.broadcasted_iota(jnp.int32, (PR, 128), 1))

    def tk_body(i, u):
        m = jnp.max(u)
        idx = jnp.min(jnp.where(u == m, fi, PS))
        val = jnp.tanh(m / pn)
        xt_ref[pl.ds(i, 1), :] = x_ref[pl.ds(idx, 1), :] * val
        return jnp.where(fi == idx, _SENT, u)

    lax.fori_loop(0, D, tk_body, u2_ref[...])

    x = xt_ref[...]
    h = h0_ref[...]
    gi = jnp.dot(x, wihT_ref[...], preferred_element_type=jnp.float32) \
        + bih_ref[...]
    gh = jnp.dot(h, whhT_ref[...], preferred_element_type=jnp.float32) \
        + bhh_ref[...]
    r = jax.nn.sigmoid(gi[:, :D] + gh[:, :D])
    z = jax.nn.sigmoid(gi[:, D:2 * D] + gh[:, D:2 * D])
    n = jnp.tanh(gi[:, 2 * D:] + r * gh[:, 2 * D:])
    wt = (1.0 - z) * n + z * h

    xw_ref[...] = jnp.dot(x_ref[...], wt, preferred_element_type=jnp.float32)


def _k2b(u2, X, pn, h0, wihT, whhT, bih2, bhh2):
    return pl.pallas_call(
        _evolve_body,
        out_shape=jax.ShapeDtypeStruct((N, D), jnp.float32),
        scratch_shapes=[pltpu.VMEM((D, D), jnp.float32)],
    )(u2, X, pn, h0, wihT, whhT, bih2, bhh2)


def _scale_body(xw_ref, degs_ref, y_ref, dinv_ref):
    dinv = lax.rsqrt(1.0 + degs_ref[...])
    dinv_ref[...] = dinv
    y_ref[...] = xw_ref[...] * dinv


def _k2y(XW, degs):
    return pl.pallas_call(
        _scale_body,
        out_shape=[
            jax.ShapeDtypeStruct((N, D), jnp.float32),
            jax.ShapeDtypeStruct((N, 1), jnp.float32),
        ],
    )(XW, degs)


def _edge_body(roww_hbm, cole_hbm, y_hbm, zeros_hbm, out_hbm,
               ridx_v, cidx_v, cadj_v, rows_v, zbuf_v, gsem, shared_acc):
    c = lax.axis_index("c")
    s = lax.axis_index("s")
    lo = c * NH
    base = s * ZSH
    pltpu.sync_copy(zeros_hbm, zbuf_v)
    for k in range(NZH):
        pltpu.sync_copy(zbuf_v, shared_acc.at[pl.ds(base + k * ZRH, ZRH)])
    plsc.subcore_barrier()

    def body(j, carry):
        cp = pltpu.async_copy(y_hbm.at[ridx_v.at[j]], rows_v, gsem)
        _remap(cidx_v, cadj_v, j, lo)
        cp.wait()
        pltpu.sync_copy(rows_v, shared_acc.at[cadj_v], add=True)
        return carry

    for seg in (s, s + NS):
        pltpu.sync_copy(roww_hbm.at[seg], ridx_v)
        pltpu.sync_copy(cole_hbm.at[seg], cidx_v)
        lax.fori_loop(0, NCHUNK, body, 0)
    plsc.subcore_barrier()
    for k in range(NZH):
        pltpu.sync_copy(shared_acc.at[pl.ds(base + k * ZRH, ZRH)],
                        out_hbm.at[c, pl.ds(base + k * ZRH, ZRH)])


def _k4(roww, cole, Y, zeros_acc):
    mesh = plsc.VectorSubcoreMesh(core_axis_name="c", subcore_axis_name="s")
    return pl.kernel(
        _edge_body,
        out_type=jax.ShapeDtypeStruct((NC, NACC, D), jnp.float32),
        mesh=mesh,
        scratch_types=[
            pltpu.VMEM((NCHUNK, CH), jnp.int32),
            pltpu.VMEM((EPP,), jnp.int32),
            pltpu.VMEM((CH,), jnp.int32),
            pltpu.VMEM((CH, D), jnp.float32),
            pltpu.VMEM((ZRH, D), jnp.float32),
            pltpu.SemaphoreType.DMA,
            pltpu.VMEM_SHARED((NACC, D), jnp.float32),
        ],
    )(roww, cole, Y, zeros_acc)


def _final_body(acc_ref, y_ref, dinv_ref, out_ref):
    for p in (0, 1):
        sl = pl.ds(p * NH, NH)
        t = (acc_ref[p, :NH, :] + y_ref[sl, :]) * dinv_ref[sl, :]
        h = jnp.maximum(t, 0.0)
        nrm = jnp.sqrt(jnp.sum(h * h, axis=1, keepdims=True))
        out_ref[sl, :] = h / jnp.maximum(nrm, 1e-12)


def _k5(acc, Y, dinv):
    return pl.pallas_call(
        _final_body,
        out_shape=jax.ShapeDtypeStruct((N, D), jnp.float32),
    )(acc, Y, dinv)


def kernel(H_K_prev, edgelists, initial_weight, p, W_ih, W_hh, b_ih, b_hh):
    X = H_K_prev[0]
    row = edgelists[0, 0].reshape(NW, NCHUNK, CH)
    cole = edgelists[0, 1].reshape(NW, EPW)

    u, pn = _k2a(X, p.reshape(D, 1))
    u2 = jnp.concatenate(
        [u.reshape(N), jnp.full((PS - N,), _SENT, jnp.float32)]).reshape(PR, 128)

    zacc = jnp.zeros((ZRH, D), jnp.float32)
    dg = _k1(cole, zacc)
    XW = _k2b(u2, X, pn, initial_weight[0],
              W_ih.T, W_hh.T,
              b_ih.reshape(1, 3 * D), b_hh.reshape(1, 3 * D))

    degs = jnp.concatenate([dg[0, :NH, 0], dg[1, :NH, 0]]).reshape(N, 1)
    Y, dinv = _k2y(XW, degs)
    acc = _k4(row, cole, Y, zacc)
    out = _k5(acc, Y, dinv)
    return out[None, :, :]


# speedup vs baseline: 1.2713x; 1.2713x over previous
import jax
import jax.numpy as jnp
from jax import lax
from jax.experimental import pallas as pl
from jax.experimental.pallas import tpu as pltpu
from jax.experimental.pallas import tpu_sc as plsc

N = 10000
D = 128
E = 320000
PS = 10240
PR = PS // 128

NC = 2
NS = 16
NW = NC * NS
EPW = E // NW
CH = 80
NCHUNK = EPW // CH
EPP = NCHUNK * CH
NH = N // 2
NACC = NH + 8
GROW = NH
ZRH = 112
NZH = 4
ZSH = 304

_SENT = -3.0e38


def _score_body(x_ref, p_ref, u_ref, pn_ref):
    p = p_ref[...]
    pn_ref[...] = jnp.sqrt(
        jnp.sum(p * p, axis=(0, 1), keepdims=True)) + 1e-16
    u_ref[...] = jnp.dot(x_ref[...], p, preferred_element_type=jnp.float32)


def _k2a(X, p2):
    return pl.pallas_call(
        _score_body,
        out_shape=[
            jax.ShapeDtypeStruct((N, 1), jnp.float32),
            jax.ShapeDtypeStruct((1, 1), jnp.float32),
        ],
    )(X, p2)


def _remap(cidx_v, cadj_v, j, lo):
    for g in range(CH // 16):
        v = cidx_v[pl.ds(j * CH + g * 16, 16)] - lo
        ok = (v >= 0) & (v < NH)
        cadj_v[pl.ds(g * 16, 16)] = jnp.where(ok, v, GROW)


def _deg_body(cole_hbm, zeros_hbm, out_hbm, cidx_v, cadj_v, ones_v,
              zbuf_v, shared_deg):
    c = lax.axis_index("c")
    s = lax.axis_index("s")
    lo = c * NH
    base = s * ZSH
    one16 = jnp.ones((16,), jnp.float32)
    for i in range(CH):
        for t in range(D // 16):
            ones_v[i, pl.ds(t * 16, 16)] = one16
    pltpu.sync_copy(zeros_hbm, zbuf_v)
    for k in range(NZH):
        pltpu.sync_copy(zbuf_v, shared_deg.at[pl.ds(base + k * ZRH, ZRH)])
    plsc.subcore_barrier()

    def body(j, carry):
        _remap(cidx_v, cadj_v, j, lo)
        pltpu.sync_copy(ones_v, shared_deg.at[cadj_v], add=True)
        return carry

    for seg in (s, s + NS):
        pltpu.sync_copy(cole_hbm.at[seg], cidx_v)
        lax.fori_loop(0, NCHUNK, body, 0)
    plsc.subcore_barrier()
    for k in range(NZH):
        pltpu.sync_copy(shared_deg.at[pl.ds(base + k * ZRH, ZRH)],
                        out_hbm.at[c, pl.ds(base + k * ZRH, ZRH)])


def _k1(cole, zeros_deg):
    mesh = plsc.VectorSubcoreMesh(core_axis_name="c", subcore_axis_name="s")
    return pl.kernel(
        _deg_body,
        out_type=jax.ShapeDtypeStruct((NC, NACC, D), jnp.float32),
        mesh=mesh,
        scratch_types=[
            pltpu.VMEM((EPP,), jnp.int32),
            pltpu.VMEM((CH,), jnp.int32),
            pltpu.VMEM((CH, D), jnp.float32),
            pltpu.VMEM((ZRH, D), jnp.float32),
            pltpu.VMEM_SHARED((NACC, D), jnp.float32),
        ],
    )(cole, zeros_deg)


def _evolve_body(u2_ref, x_ref, pn_ref, h0_ref, wihT_ref, whhT_ref,
                 bih_ref, bhh_ref, xw_ref, xt_ref):
    pn = pn_ref[0, 0]
    fi = (lax.broadcasted_iota(jnp.int32, (PR, 128), 0) * 128 +
          lax.broadcasted_iota(jnp.int32, (PR, 128), 1))

    def tk_body(i, u):
        m = jnp.max(u)
        idx = jnp.min(jnp.where(u == m, fi, PS))
        val = jnp.tanh(m / pn)
        xt_ref[pl.ds(i, 1), :] = x_ref[pl.ds(idx, 1), :] * val
        return jnp.where(fi == idx, _SENT, u)

    lax.fori_loop(0, D, tk_body, u2_ref[...])

    x = xt_ref[...]
    h = h0_ref[...]
    gi = jnp.dot(x, wihT_ref[...], preferred_element_type=jnp.float32) \
        + bih_ref[...]
    gh = jnp.dot(h, whhT_ref[...], preferred_element_type=jnp.float32) \
        + bhh_ref[...]
    r = jax.nn.sigmoid(gi[:, :D] + gh[:, :D])
    z = jax.nn.sigmoid(gi[:, D:2 * D] + gh[:, D:2 * D])
    n = jnp.tanh(gi[:, 2 * D:] + r * gh[:, 2 * D:])
    wt = (1.0 - z) * n + z * h

    xw_ref[...] = jnp.dot(x_ref[...], wt, preferred_element_type=jnp.float32)


def _k2b(u2, X, pn, h0, wihT, whhT, bih2, bhh2):
    return pl.pallas_call(
        _evolve_body,
        out_shape=jax.ShapeDtypeStruct((N, D), jnp.float32),
        scratch_shapes=[pltpu.VMEM((D, D), jnp.float32)],
    )(u2, X, pn, h0, wihT, whhT, bih2, bhh2)


def _scale_body(xw_ref, degs_ref, y_ref, dinv_ref):
    dinv = lax.rsqrt(1.0 + degs_ref[...])
    dinv_ref[...] = dinv
    y_ref[...] = xw_ref[...] * dinv


def _k2y(XW, degs):
    return pl.pallas_call(
        _scale_body,
        out_shape=[
            jax.ShapeDtypeStruct((N, D), jnp.float32),
            jax.ShapeDtypeStruct((N, 1), jnp.float32),
        ],
    )(XW, degs)


def _edge_body(roww_hbm, cole_hbm, y_hbm, zeros_hbm, out_hbm,
               ridx_v, cidx_v, cadj0_v, cadj1_v, rows0_v, rows1_v,
               zbuf_v, gsem0, gsem1, shared_acc):
    c = lax.axis_index("c")
    s = lax.axis_index("s")
    lo = c * NH
    base = s * ZSH
    pltpu.sync_copy(zeros_hbm, zbuf_v)
    for k in range(NZH):
        pltpu.sync_copy(zbuf_v, shared_acc.at[pl.ds(base + k * ZRH, ZRH)])
    plsc.subcore_barrier()

    def half(j, cadj_v, rows_v, gsem):
        _remap(cidx_v, cadj_v, j, lo)
        pltpu.make_async_copy(y_hbm.at[ridx_v.at[j]], rows_v, gsem).wait()
        pltpu.sync_copy(rows_v, shared_acc.at[cadj_v], add=True)

    def pair(t, carry):
        j0 = 2 * t
        j1 = j0 + 1
        half(j0, cadj0_v, rows0_v, gsem0)

        @pl.when(j0 + 2 < NCHUNK)
        def _():
            pltpu.async_copy(y_hbm.at[ridx_v.at[j0 + 2]], rows0_v, gsem0)

        half(j1, cadj1_v, rows1_v, gsem1)

        @pl.when(j1 + 2 < NCHUNK)
        def _():
            pltpu.async_copy(y_hbm.at[ridx_v.at[j1 + 2]], rows1_v, gsem1)

        return carry

    for seg in (s, s + NS):
        pltpu.sync_copy(roww_hbm.at[seg], ridx_v)
        pltpu.sync_copy(cole_hbm.at[seg], cidx_v)
        pltpu.async_copy(y_hbm.at[ridx_v.at[0]], rows0_v, gsem0)
        pltpu.async_copy(y_hbm.at[ridx_v.at[1]], rows1_v, gsem1)
        lax.fori_loop(0, NCHUNK // 2, pair, 0)
        half(NCHUNK - 1, cadj0_v, rows0_v, gsem0)
    plsc.subcore_barrier()
    for k in range(NZH):
        pltpu.sync_copy(shared_acc.at[pl.ds(base + k * ZRH, ZRH)],
                        out_hbm.at[c, pl.ds(base + k * ZRH, ZRH)])


def _k4(roww, cole, Y, zeros_acc):
    mesh = plsc.VectorSubcoreMesh(core_axis_name="c", subcore_axis_name="s")
    return pl.kernel(
        _edge_body,
        out_type=jax.ShapeDtypeStruct((NC, NACC, D), jnp.float32),
        mesh=mesh,
        scratch_types=[
            pltpu.VMEM((NCHUNK, CH), jnp.int32),
            pltpu.VMEM((EPP,), jnp.int32),
            pltpu.VMEM((CH,), jnp.int32),
            pltpu.VMEM((CH,), jnp.int32),
            pltpu.VMEM((CH, D), jnp.float32),
            pltpu.VMEM((CH, D), jnp.float32),
            pltpu.VMEM((ZRH, D), jnp.float32),
            pltpu.SemaphoreType.DMA,
            pltpu.SemaphoreType.DMA,
            pltpu.VMEM_SHARED((NACC, D), jnp.float32),
        ],
    )(roww, cole, Y, zeros_acc)


def _final_body(acc_ref, y_ref, dinv_ref, out_ref):
    for p in (0, 1):
        sl = pl.ds(p * NH, NH)
        t = (acc_ref[p, :NH, :] + y_ref[sl, :]) * dinv_ref[sl, :]
        h = jnp.maximum(t, 0.0)
        nrm = jnp.sqrt(jnp.sum(h * h, axis=1, keepdims=True))
        out_ref[sl, :] = h / jnp.maximum(nrm, 1e-12)


def _k5(acc, Y, dinv):
    return pl.pallas_call(
        _final_body,
        out_shape=jax.ShapeDtypeStruct((N, D), jnp.float32),
    )(acc, Y, dinv)


def kernel(H_K_prev, edgelists, initial_weight, p, W_ih, W_hh, b_ih, b_hh):
    X = H_K_prev[0]
    row = edgelists[0, 0].reshape(NW, NCHUNK, CH)
    cole = edgelists[0, 1].reshape(NW, EPW)

    u, pn = _k2a(X, p.reshape(D, 1))
    u2 = jnp.concatenate(
        [u.reshape(N), jnp.full((PS - N,), _SENT, jnp.float32)]).reshape(PR, 128)

    zacc = jnp.zeros((ZRH, D), jnp.float32)
    dg = _k1(cole, zacc)
    XW = _k2b(u2, X, pn, initial_weight[0],
              W_ih.T, W_hh.T,
              b_ih.reshape(1, 3 * D), b_hh.reshape(1, 3 * D))

    degs = jnp.concatenate([dg[0, :NH, 0], dg[1, :NH, 0]]).reshape(N, 1)
    Y, dinv = _k2y(XW, degs)
    acc = _k4(row, cole, Y, zacc)
    out = _k5(acc, Y, dinv)
    return out[None, :, :]
